# baseline (speedup 1.0000x reference)
import functools

import jax
import jax.numpy as jnp
from jax import lax
from jax.experimental import pallas as pl
from jax.experimental.pallas import tpu as pltpu
from jax.experimental.pallas import tpu_sc as plsc

N = 10000
E = 320000
D = 128
H = 64
O = 128

NC = 2
NS = 16
NW = NC * NS
CHUNK = 125
NGRP = 2
GCHUNK = 40
ZB = 80
NP = 10240
ROWS_PER_TILE = NP // NS


def _make_sc_scatter():
    mesh = plsc.VectorSubcoreMesh(core_axis_name="c", subcore_axis_name="s",
                                  num_cores=NC, num_subcores=NS)

    @functools.partial(
        pl.kernel,
        out_type=jax.ShapeDtypeStruct((NC, NP, D), jnp.float32),
        mesh=mesh,
        scratch_types=[
            pltpu.VMEM((GCHUNK, CHUNK), jnp.int32),
            pltpu.VMEM((GCHUNK, CHUNK), jnp.int32),
            pltpu.VMEM((CHUNK, D), jnp.float32),
            pltpu.VMEM((CHUNK, D), jnp.float32),
            pltpu.VMEM_SHARED((NP, D), jnp.float32),
            pltpu.SemaphoreType.DMA,
            pltpu.SemaphoreType.DMA,
            pltpu.SemaphoreType.DMA,
            pltpu.SemaphoreType.DMA,
        ],
    )
    def sc_scatter(x_hbm, src_hbm, dst_hbm, dummy_hbm, out_hbm,
                   src_v, dst_v, rows0, rows1, ax_sp, sem0, sem1,
                   ssem0, ssem1):
        c = lax.axis_index("c")
        s = lax.axis_index("s")
        w = s * NC + c

        zero16 = jnp.zeros((16,), jnp.float32)

        def zrow(r, carry):
            for k in range(D // 16):
                rows0[r, pl.ds(k * 16, 16)] = zero16
            return carry

        lax.fori_loop(0, ZB, zrow, 0)
        for t in range(ROWS_PER_TILE // ZB):
            pltpu.async_copy(rows0.at[pl.ds(0, ZB)],
                             ax_sp.at[pl.ds(s * ROWS_PER_TILE + t * ZB, ZB)],
                             sem0)
        for t in range(ROWS_PER_TILE // ZB):
            pltpu.make_async_copy(
                rows0.at[pl.ds(0, ZB)],
                ax_sp.at[pl.ds(s * ROWS_PER_TILE + t * ZB, ZB)],
                sem0).wait()
        plsc.subcore_barrier()

        for g in range(NGRP):
            pltpu.sync_copy(src_hbm.at[w, g], src_v)
            pltpu.sync_copy(dst_hbm.at[w, g], dst_v)

            pltpu.async_copy(x_hbm.at[src_v.at[0]], rows0, sem0)
            pltpu.async_copy(x_hbm.at[src_v.at[1]], rows1, sem1)

            def step(i, carry):
                j = 2 * i
                pltpu.make_async_copy(x_hbm.at[src_v.at[j]], rows0,
                                      sem0).wait()
                pltpu.async_copy(rows0, ax_sp.at[dst_v.at[j]], ssem0,
                                 add=True)
                pltpu.make_async_copy(x_hbm.at[src_v.at[j + 1]], rows1,
                                      sem1).wait()
                pltpu.async_copy(rows1, ax_sp.at[dst_v.at[j + 1]], ssem1,
                                 add=True)
                pltpu.make_async_copy(dummy_hbm, rows0, ssem0).wait()

                @pl.when(j + 2 < GCHUNK)
                def _():
                    pltpu.async_copy(x_hbm.at[src_v.at[j + 2]], rows0, sem0)

                pltpu.make_async_copy(dummy_hbm, rows1, ssem1).wait()

                @pl.when(j + 3 < GCHUNK)
                def _():
                    pltpu.async_copy(x_hbm.at[src_v.at[j + 3]], rows1, sem1)

                return carry

            lax.fori_loop(0, GCHUNK // 2, step, 0)
        plsc.subcore_barrier()

        pltpu.sync_copy(ax_sp.at[pl.ds(s * ROWS_PER_TILE, ROWS_PER_TILE)],
                        out_hbm.at[c, pl.ds(s * ROWS_PER_TILE, ROWS_PER_TILE)])

    return sc_scatter


_sc_scatter = _make_sc_scatter()

ROW_BLK = 2000


def _mlp_body(a0_ref, a1_ref, w1_ref, b1_ref, w2_ref, b2_ref, o_ref):
    ax = a0_ref[...] + a1_ref[...]
    h = jnp.dot(ax, w1_ref[...], preferred_element_type=jnp.float32)
    h = jnp.maximum(h + b1_ref[...], 0.0)
    o_ref[...] = jnp.dot(h, w2_ref[...],
                         preferred_element_type=jnp.float32) + b2_ref[...]


def _mlp(a0, a1, W1, b1, W2, b2):
    return pl.pallas_call(
        _mlp_body,
        grid=(N // ROW_BLK,),
        in_specs=[
            pl.BlockSpec((ROW_BLK, D), lambda i: (i, 0)),
            pl.BlockSpec((ROW_BLK, D), lambda i: (i, 0)),
            pl.BlockSpec((D, H), lambda i: (0, 0)),
            pl.BlockSpec((1, H), lambda i: (0, 0)),
            pl.BlockSpec((H, O), lambda i: (0, 0)),
            pl.BlockSpec((1, O), lambda i: (0, 0)),
        ],
        out_specs=pl.BlockSpec((ROW_BLK, O), lambda i: (i, 0)),
        out_shape=jax.ShapeDtypeStruct((N, O), jnp.float32),
    )(a0, a1, W1, b1, W2, b2)


def kernel(x, src, dst, W1, b1, W2, b2):
    src_i = src.astype(jnp.int32).reshape(NW, NGRP, GCHUNK, CHUNK)
    dst_i = dst.astype(jnp.int32).reshape(NW, NGRP, GCHUNK, CHUNK)
    dummy = jnp.zeros((CHUNK, D), jnp.float32)
    partials = _sc_scatter(x, src_i, dst_i, dummy)
    return _mlp(partials[0], partials[1], W1,
                b1.reshape(1, H), W2, b2.reshape(1, O))


# speedup vs baseline: 1.2498x; 1.2498x over previous
import functools

import jax
import jax.numpy as jnp
from jax import lax
from jax.experimental import pallas as pl
from jax.experimental.pallas import tpu as pltpu
from jax.experimental.pallas import tpu_sc as plsc

N = 10000
E = 320000
D = 128
H = 64
O = 128

NC = 2
NS = 16
NW = NC * NS
CHUNK = 125
NGRP = 2
GCHUNK = 40
ZB = 80
NP = 10240
ROWS_PER_TILE = NP // NS


def _make_sc_scatter():
    mesh = plsc.VectorSubcoreMesh(core_axis_name="c", subcore_axis_name="s",
                                  num_cores=NC, num_subcores=NS)

    @functools.partial(
        pl.kernel,
        out_type=jax.ShapeDtypeStruct((NC, NP, D), jnp.float32),
        mesh=mesh,
        scratch_types=[
            pltpu.VMEM((GCHUNK, CHUNK), jnp.int32),
            pltpu.VMEM((GCHUNK, CHUNK), jnp.int32),
            pltpu.VMEM((CHUNK, D), jnp.float32),
            pltpu.VMEM((CHUNK, D), jnp.float32),
            pltpu.VMEM_SHARED((NP, D), jnp.float32),
            pltpu.SemaphoreType.DMA,
            pltpu.SemaphoreType.DMA,
        ],
    )
    def sc_scatter(x_hbm, src_hbm, dst_hbm, out_hbm,
                   src_v, dst_v, rows0, rows1, ax_sp, sem0, sem1):
        c = lax.axis_index("c")
        s = lax.axis_index("s")
        w = s * NC + c

        zero16 = jnp.zeros((16,), jnp.float32)

        def zrow(r, carry):
            for k in range(D // 16):
                rows0[r, pl.ds(k * 16, 16)] = zero16
            return carry

        lax.fori_loop(0, ZB, zrow, 0)
        for t in range(ROWS_PER_TILE // ZB):
            pltpu.async_copy(rows0.at[pl.ds(0, ZB)],
                             ax_sp.at[pl.ds(s * ROWS_PER_TILE + t * ZB, ZB)],
                             sem0)
        for t in range(ROWS_PER_TILE // ZB):
            pltpu.make_async_copy(
                rows0.at[pl.ds(0, ZB)],
                ax_sp.at[pl.ds(s * ROWS_PER_TILE + t * ZB, ZB)],
                sem0).wait()
        plsc.subcore_barrier()

        for g in range(NGRP):
            pltpu.sync_copy(src_hbm.at[w, g], src_v)
            pltpu.sync_copy(dst_hbm.at[w, g], dst_v)

            pltpu.async_copy(x_hbm.at[src_v.at[0]], rows0, sem0)
            pltpu.async_copy(x_hbm.at[src_v.at[1]], rows1, sem1)

            def step(i, carry):
                j = 2 * i
                pltpu.make_async_copy(x_hbm.at[src_v.at[j]], rows0,
                                      sem0).wait()
                pltpu.sync_copy(rows0, ax_sp.at[dst_v.at[j]], add=True)

                @pl.when(j + 2 < GCHUNK)
                def _():
                    pltpu.async_copy(x_hbm.at[src_v.at[j + 2]], rows0, sem0)

                pltpu.make_async_copy(x_hbm.at[src_v.at[j + 1]], rows1,
                                      sem1).wait()
                pltpu.sync_copy(rows1, ax_sp.at[dst_v.at[j + 1]], add=True)

                @pl.when(j + 3 < GCHUNK)
                def _():
                    pltpu.async_copy(x_hbm.at[src_v.at[j + 3]], rows1, sem1)

                return carry

            lax.fori_loop(0, GCHUNK // 2, step, 0)
        plsc.subcore_barrier()

        pltpu.sync_copy(ax_sp.at[pl.ds(s * ROWS_PER_TILE, ROWS_PER_TILE)],
                        out_hbm.at[c, pl.ds(s * ROWS_PER_TILE, ROWS_PER_TILE)])

    return sc_scatter


_sc_scatter = _make_sc_scatter()

ROW_BLK = 2000


def _mlp_body(a0_ref, a1_ref, w1_ref, b1_ref, w2_ref, b2_ref, o_ref):
    ax = a0_ref[...] + a1_ref[...]
    h = jnp.dot(ax, w1_ref[...], preferred_element_type=jnp.float32)
    h = jnp.maximum(h + b1_ref[...], 0.0)
    o_ref[...] = jnp.dot(h, w2_ref[...],
                         preferred_element_type=jnp.float32) + b2_ref[...]


def _mlp(a0, a1, W1, b1, W2, b2):
    return pl.pallas_call(
        _mlp_body,
        grid=(N // ROW_BLK,),
        in_specs=[
            pl.BlockSpec((ROW_BLK, D), lambda i: (i, 0)),
            pl.BlockSpec((ROW_BLK, D), lambda i: (i, 0)),
            pl.BlockSpec((D, H), lambda i: (0, 0)),
            pl.BlockSpec((1, H), lambda i: (0, 0)),
            pl.BlockSpec((H, O), lambda i: (0, 0)),
            pl.BlockSpec((1, O), lambda i: (0, 0)),
        ],
        out_specs=pl.BlockSpec((ROW_BLK, O), lambda i: (i, 0)),
        out_shape=jax.ShapeDtypeStruct((N, O), jnp.float32),
    )(a0, a1, W1, b1, W2, b2)


def kernel(x, src, dst, W1, b1, W2, b2):
    src_i = src.astype(jnp.int32).reshape(NW, NGRP, GCHUNK, CHUNK)
    dst_i = dst.astype(jnp.int32).reshape(NW, NGRP, GCHUNK, CHUNK)
    partials = _sc_scatter(x, src_i, dst_i)
    return _mlp(partials[0], partials[1], W1,
                b1.reshape(1, H), W2, b2.reshape(1, O))
